# baseline (speedup 1.0000x reference)
import functools

import jax
import jax.numpy as jnp
from jax import lax
from jax.experimental import pallas as pl
from jax.experimental.pallas import tpu as pltpu
from jax.experimental.pallas import tpu_sc as plsc

T = 4096
D = 1024
E = 8
I = 512
CAP = 1024
NSLOT = E * CAP
NSLOT_PAD = 8448
NB = 8
BT = T // NB
NW = 32
ROWS_W = NSLOT_PAD // NW
DCHUNK = 88
TOK_W = T // NW
CCHUNK = 32


def _router_body(x_ref, wg_ref, ti0_ref, ti1_ref, p0_ref, p1_ref,
                 w0_ref, w1_ref, cnt0_ref, laux_ref, xbf_ref,
                 off0, off1, me_acc):
    b = pl.program_id(0)

    @pl.when(b == 0)
    def _init():
        off0[...] = jnp.zeros_like(off0)
        off1[...] = jnp.zeros_like(off1)
        me_acc[...] = jnp.zeros_like(me_acc)

    xb = x_ref[...]
    wg = wg_ref[...]
    logits = jnp.dot(xb, wg, preferred_element_type=jnp.float32)
    m = jnp.max(logits, axis=1, keepdims=True)
    ex = jnp.exp(logits - m)
    gates = ex / jnp.sum(ex, axis=1, keepdims=True)

    lane = lax.broadcasted_iota(jnp.int32, (BT, E), 1)
    v0 = jnp.max(gates, axis=1, keepdims=True)
    i0 = jnp.min(jnp.where(gates == v0, lane, E), axis=1, keepdims=True)
    g1 = jnp.where(lane == i0, -jnp.inf, gates)
    v1 = jnp.max(g1, axis=1, keepdims=True)
    i1 = jnp.min(jnp.where(g1 == v1, lane, E), axis=1, keepdims=True)
    denom = v0 + v1 + 1e-9
    mask0 = (lane == i0).astype(jnp.float32)
    mask1 = (lane == i1).astype(jnp.float32)

    tri = (lax.broadcasted_iota(jnp.int32, (BT, BT), 0)
           >= lax.broadcasted_iota(jnp.int32, (BT, BT), 1)).astype(jnp.float32)
    c0 = jnp.dot(tri, mask0, preferred_element_type=jnp.float32)
    c1 = jnp.dot(tri, mask1, preferred_element_type=jnp.float32)

    o0 = off0[...]
    o1 = off1[...]
    pos0 = jnp.sum(mask0 * (c0 - 1.0 + o0), axis=1, keepdims=True)
    pos1 = jnp.sum(mask1 * (c1 - 1.0 + o1), axis=1, keepdims=True)
    new_off0 = o0 + jnp.sum(mask0, axis=0, keepdims=True)
    off0[...] = new_off0
    off1[...] = o1 + jnp.sum(mask1, axis=0, keepdims=True)
    me_acc[...] = me_acc[...] + jnp.sum(gates, axis=0, keepdims=True)

    ti0_ref[...] = i0.astype(jnp.int32).reshape(1, BT, 1)
    ti1_ref[...] = i1.astype(jnp.int32).reshape(1, BT, 1)
    p0_ref[...] = pos0.astype(jnp.int32).reshape(1, BT, 1)
    p1_ref[...] = pos1.astype(jnp.int32).reshape(1, BT, 1)
    w0_ref[...] = (v0 / denom).reshape(1, BT, 1)
    w1_ref[...] = (v1 / denom).reshape(1, BT, 1)
    cnt0_ref[...] = new_off0.astype(jnp.int32)
    xbf_ref[...] = xb.astype(jnp.bfloat16)

    @pl.when(b == NB - 1)
    def _fin():
        me = me_acc[...] / float(T)
        ce = new_off0 / float(T)
        laux_ref[...] = jnp.sum(me * ce).reshape(1, 1) * float(E)


def _router(x, wg, interpret=False):
    out_shapes = (
        jax.ShapeDtypeStruct((NB, BT, 1), jnp.int32),
        jax.ShapeDtypeStruct((NB, BT, 1), jnp.int32),
        jax.ShapeDtypeStruct((NB, BT, 1), jnp.int32),
        jax.ShapeDtypeStruct((NB, BT, 1), jnp.int32),
        jax.ShapeDtypeStruct((NB, BT, 1), jnp.float32),
        jax.ShapeDtypeStruct((NB, BT, 1), jnp.float32),
        jax.ShapeDtypeStruct((1, E), jnp.int32),
        jax.ShapeDtypeStruct((1, 1), jnp.float32),
        jax.ShapeDtypeStruct((T, D), jnp.bfloat16),
    )
    blk = pl.BlockSpec((1, BT, 1), lambda i: (i, 0, 0))
    return pl.pallas_call(
        _router_body,
        grid=(NB,),
        in_specs=[
            pl.BlockSpec((BT, D), lambda i: (i, 0)),
            pl.BlockSpec((D, E), lambda i: (0, 0)),
        ],
        out_specs=(blk, blk, blk, blk, blk, blk,
                   pl.BlockSpec((1, E), lambda i: (0, 0)),
                   pl.BlockSpec((1, 1), lambda i: (0, 0)),
                   pl.BlockSpec((BT, D), lambda i: (i, 0))),
        out_shape=out_shapes,
        scratch_shapes=[
            pltpu.VMEM((1, E), jnp.float32),
            pltpu.VMEM((1, E), jnp.float32),
            pltpu.VMEM((1, E), jnp.float32),
        ],
        interpret=interpret,
    )(x, wg)


def _slot_build_body(ti0_hbm, ti1_hbm, p0_hbm, p1_hbm, w0_hbm, w1_hbm,
                     cnt0_hbm, s2t_hbm, sw_hbm, ci0_hbm, ci1_hbm,
                     ti0_v, ti1_v, p0_v, p1_v, w0_v, w1_v, cnt0_v,
                     s2t_v, sw_v, ci0_v, ci1_v):
    wid = lax.axis_index("s") * 2 + lax.axis_index("c")

    @pl.when(wid == 0)
    def _():
        pltpu.sync_copy(ti0_hbm, ti0_v)
        pltpu.sync_copy(ti1_hbm, ti1_v)
        pltpu.sync_copy(p0_hbm, p0_v)
        pltpu.sync_copy(p1_hbm, p1_v)
        pltpu.sync_copy(w0_hbm, w0_v)
        pltpu.sync_copy(w1_hbm, w1_v)
        pltpu.sync_copy(cnt0_hbm, cnt0_v)

        zi = jnp.zeros((16,), jnp.int32)
        zf = jnp.zeros((16,), jnp.float32)

        def zloop(i, carry):
            s2t_v[pl.ds(i * 16, 16)] = zi
            sw_v[pl.ds(i * 16, 16)] = zf
            return carry

        lax.fori_loop(0, NSLOT_PAD // 16, zloop, 0)

        def tloop(i, carry):
            sl = pl.ds(i * 16, 16)
            t0 = ti0_v[sl]
            t1 = ti1_v[sl]
            pos0 = p0_v[sl]
            pos1 = p1_v[sl] + plsc.load_gather(cnt0_v, [t1])
            tok = i * 16 + lax.iota(jnp.int32, 16)
            k0 = pos0 < CAP
            k1 = pos1 < CAP
            d0 = t0 * CAP + pos0
            d1 = t1 * CAP + pos1
            d0c = jnp.where(k0, d0, 0)
            d1c = jnp.where(k1, d1, 0)
            plsc.store_scatter(s2t_v, [d0c], tok, mask=k0)
            plsc.store_scatter(sw_v, [d0c], w0_v[sl], mask=k0)
            plsc.store_scatter(s2t_v, [d1c], tok, mask=k1)
            plsc.store_scatter(sw_v, [d1c], w1_v[sl], mask=k1)
            ci0_v[sl] = jnp.where(k0, d0, NSLOT)
            ci1_v[sl] = jnp.where(k1, d1, NSLOT)
            return carry

        lax.fori_loop(0, T // 16, tloop, 0)

        pltpu.sync_copy(s2t_v, s2t_hbm)
        pltpu.sync_copy(sw_v, sw_hbm)
        pltpu.sync_copy(ci0_v, ci0_hbm)
        pltpu.sync_copy(ci1_v, ci1_hbm)


def _dispatch_body(x_hbm, s2t_hbm, disp_hbm, idx_v, rows_v, sem):
    wid = lax.axis_index("s") * 2 + lax.axis_index("c")
    for c in range(ROWS_W // DCHUNK):
        base = wid * ROWS_W + c * DCHUNK
        pltpu.sync_copy(s2t_hbm.at[pl.ds(base, DCHUNK)], idx_v)
        pltpu.async_copy(x_hbm.at[idx_v], rows_v, sem).wait()
        pltpu.sync_copy(rows_v, disp_hbm.at[pl.ds(base, DCHUNK)])


def _combine_body(eo_hbm, ci0_hbm, ci1_hbm, y_hbm, i0_v, i1_v, a_v, b_v, sem):
    wid = lax.axis_index("s") * 2 + lax.axis_index("c")
    for c in range(TOK_W // CCHUNK):
        base = wid * TOK_W + c * CCHUNK
        pltpu.sync_copy(ci0_hbm.at[pl.ds(base, CCHUNK)], i0_v)
        pltpu.sync_copy(ci1_hbm.at[pl.ds(base, CCHUNK)], i1_v)
        cp0 = pltpu.async_copy(eo_hbm.at[i0_v], a_v, sem)
        cp1 = pltpu.async_copy(eo_hbm.at[i1_v], b_v, sem)
        cp0.wait()
        cp1.wait()

        def radd(r, carry):
            for j in range(D // 16):
                sl = pl.ds(j * 16, 16)
                a_v[r, sl] = a_v[r, sl] + b_v[r, sl]
            return carry

        lax.fori_loop(0, CCHUNK, radd, 0)
        pltpu.sync_copy(a_v, y_hbm.at[pl.ds(base, CCHUNK)])


@functools.lru_cache(maxsize=None)
def _sc_kernels():
    mesh = plsc.VectorSubcoreMesh(core_axis_name="c", subcore_axis_name="s")

    slot_build = pl.kernel(
        _slot_build_body,
        out_type=(
            jax.ShapeDtypeStruct((NSLOT_PAD,), jnp.int32),
            jax.ShapeDtypeStruct((NSLOT_PAD,), jnp.float32),
            jax.ShapeDtypeStruct((T,), jnp.int32),
            jax.ShapeDtypeStruct((T,), jnp.int32),
        ),
        mesh=mesh,
        compiler_params=pltpu.CompilerParams(needs_layout_passes=False),
        scratch_types=[
            pltpu.VMEM((T,), jnp.int32),
            pltpu.VMEM((T,), jnp.int32),
            pltpu.VMEM((T,), jnp.int32),
            pltpu.VMEM((T,), jnp.int32),
            pltpu.VMEM((T,), jnp.float32),
            pltpu.VMEM((T,), jnp.float32),
            pltpu.VMEM((16,), jnp.int32),
            pltpu.VMEM((NSLOT_PAD,), jnp.int32),
            pltpu.VMEM((NSLOT_PAD,), jnp.float32),
            pltpu.VMEM((T,), jnp.int32),
            pltpu.VMEM((T,), jnp.int32),
        ],
    )

    dispatch = pl.kernel(
        _dispatch_body,
        out_type=jax.ShapeDtypeStruct((NSLOT_PAD, D), jnp.float32),
        mesh=mesh,
        compiler_params=pltpu.CompilerParams(needs_layout_passes=False),
        scratch_types=[
            pltpu.VMEM((DCHUNK,), jnp.int32),
            pltpu.VMEM((DCHUNK, D), jnp.float32),
            pltpu.SemaphoreType.DMA,
        ],
    )

    combine = pl.kernel(
        _combine_body,
        out_type=jax.ShapeDtypeStruct((T, D), jnp.float32),
        mesh=mesh,
        compiler_params=pltpu.CompilerParams(needs_layout_passes=False),
        scratch_types=[
            pltpu.VMEM((CCHUNK,), jnp.int32),
            pltpu.VMEM((CCHUNK,), jnp.int32),
            pltpu.VMEM((CCHUNK, D), jnp.float32),
            pltpu.VMEM((CCHUNK, D), jnp.float32),
            pltpu.SemaphoreType.DMA,
        ],
    )
    return slot_build, dispatch, combine


def _bmm_body(sw_ref, disp_ref, wg_ref, wu_ref, wd_ref, out_ref):
    xb = disp_ref[...].astype(jnp.bfloat16)
    g = jnp.dot(xb, wg_ref[0], preferred_element_type=jnp.float32)
    u = jnp.dot(xb, wu_ref[0], preferred_element_type=jnp.float32)
    h = g * jax.nn.sigmoid(g) * u
    o = jnp.dot(h.astype(jnp.bfloat16), wd_ref[0],
                preferred_element_type=jnp.float32)
    out_ref[...] = o * sw_ref[...]


def _bmm(sw, disp, w_gate, w_up, w_down, interpret=False):
    nblk = NSLOT_PAD // 128
    eidx = lambda i: (jnp.minimum(i // 8, E - 1), 0, 0)
    return pl.pallas_call(
        _bmm_body,
        grid=(nblk,),
        in_specs=[
            pl.BlockSpec((128, 1), lambda i: (i, 0)),
            pl.BlockSpec((128, D), lambda i: (i, 0)),
            pl.BlockSpec((1, D, I), eidx),
            pl.BlockSpec((1, D, I), eidx),
            pl.BlockSpec((1, I, D), eidx),
        ],
        out_specs=pl.BlockSpec((128, D), lambda i: (i, 0)),
        out_shape=jax.ShapeDtypeStruct((NSLOT_PAD, D), jnp.float32),
        interpret=interpret,
    )(sw, disp, w_gate, w_up, w_down)


@jax.jit
def kernel(x, wg, w_gate, w_up, w_down):
    slot_build, dispatch, combine = _sc_kernels()
    ti0, ti1, p0, p1, w0, w1, cnt0, laux, xbf = _router(x, wg)
    s2t, sw, ci0, ci1 = slot_build(
        ti0.reshape(T), ti1.reshape(T), p0.reshape(T), p1.reshape(T),
        w0.reshape(T), w1.reshape(T),
        jnp.concatenate([cnt0.reshape(E), jnp.zeros((16 - E,), jnp.int32)]))
    disp = dispatch(x, s2t)
    eo = _bmm(sw.reshape(NSLOT_PAD, 1), disp,
              w_gate.astype(jnp.bfloat16), w_up.astype(jnp.bfloat16),
              w_down.astype(jnp.bfloat16))
    y = combine(eo, ci0, ci1)
    return y, laux.reshape(())


# speedup vs baseline: 1.0028x; 1.0028x over previous
import functools

import jax
import jax.numpy as jnp
from jax import lax
from jax.experimental import pallas as pl
from jax.experimental.pallas import tpu as pltpu
from jax.experimental.pallas import tpu_sc as plsc

T = 4096
D = 1024
E = 8
I = 512
CAP = 1024
NSLOT = E * CAP
NSLOT_PAD = 8448
NB = 8
BT = T // NB
NW = 32
ROWS_W = NSLOT_PAD // NW
DCHUNK = 88
TOK_W = T // NW
CCHUNK = 32


def _router_body(x_ref, wg_ref, ti0_ref, ti1_ref, p0_ref, p1_ref,
                 w0_ref, w1_ref, cnt0_ref, laux_ref,
                 off0, off1, me_acc):
    b = pl.program_id(0)

    @pl.when(b == 0)
    def _init():
        off0[...] = jnp.zeros_like(off0)
        off1[...] = jnp.zeros_like(off1)
        me_acc[...] = jnp.zeros_like(me_acc)

    xb = x_ref[...]
    wg = wg_ref[...]
    logits = jnp.dot(xb, wg, preferred_element_type=jnp.float32)
    m = jnp.max(logits, axis=1, keepdims=True)
    ex = jnp.exp(logits - m)
    gates = ex / jnp.sum(ex, axis=1, keepdims=True)

    lane = lax.broadcasted_iota(jnp.int32, (BT, E), 1)
    v0 = jnp.max(gates, axis=1, keepdims=True)
    i0 = jnp.min(jnp.where(gates == v0, lane, E), axis=1, keepdims=True)
    g1 = jnp.where(lane == i0, -jnp.inf, gates)
    v1 = jnp.max(g1, axis=1, keepdims=True)
    i1 = jnp.min(jnp.where(g1 == v1, lane, E), axis=1, keepdims=True)
    denom = v0 + v1 + 1e-9
    mask0 = (lane == i0).astype(jnp.float32)
    mask1 = (lane == i1).astype(jnp.float32)

    tri = (lax.broadcasted_iota(jnp.int32, (BT, BT), 0)
           >= lax.broadcasted_iota(jnp.int32, (BT, BT), 1)).astype(jnp.float32)
    c0 = jnp.dot(tri, mask0, preferred_element_type=jnp.float32)
    c1 = jnp.dot(tri, mask1, preferred_element_type=jnp.float32)

    o0 = off0[...]
    o1 = off1[...]
    pos0 = jnp.sum(mask0 * (c0 - 1.0 + o0), axis=1, keepdims=True)
    pos1 = jnp.sum(mask1 * (c1 - 1.0 + o1), axis=1, keepdims=True)
    new_off0 = o0 + jnp.sum(mask0, axis=0, keepdims=True)
    off0[...] = new_off0
    off1[...] = o1 + jnp.sum(mask1, axis=0, keepdims=True)
    me_acc[...] = me_acc[...] + jnp.sum(gates, axis=0, keepdims=True)

    ti0_ref[...] = i0.astype(jnp.int32).reshape(1, BT, 1)
    ti1_ref[...] = i1.astype(jnp.int32).reshape(1, BT, 1)
    p0_ref[...] = pos0.astype(jnp.int32).reshape(1, BT, 1)
    p1_ref[...] = pos1.astype(jnp.int32).reshape(1, BT, 1)
    w0_ref[...] = (v0 / denom).reshape(1, BT, 1)
    w1_ref[...] = (v1 / denom).reshape(1, BT, 1)
    cnt0_ref[...] = new_off0.astype(jnp.int32)

    @pl.when(b == NB - 1)
    def _fin():
        me = me_acc[...] / float(T)
        ce = new_off0 / float(T)
        laux_ref[...] = jnp.sum(me * ce).reshape(1, 1) * float(E)


def _router(x, wg, interpret=False):
    out_shapes = (
        jax.ShapeDtypeStruct((NB, BT, 1), jnp.int32),
        jax.ShapeDtypeStruct((NB, BT, 1), jnp.int32),
        jax.ShapeDtypeStruct((NB, BT, 1), jnp.int32),
        jax.ShapeDtypeStruct((NB, BT, 1), jnp.int32),
        jax.ShapeDtypeStruct((NB, BT, 1), jnp.float32),
        jax.ShapeDtypeStruct((NB, BT, 1), jnp.float32),
        jax.ShapeDtypeStruct((1, E), jnp.int32),
        jax.ShapeDtypeStruct((1, 1), jnp.float32),
    )
    blk = pl.BlockSpec((1, BT, 1), lambda i: (i, 0, 0))
    return pl.pallas_call(
        _router_body,
        grid=(NB,),
        in_specs=[
            pl.BlockSpec((BT, D), lambda i: (i, 0)),
            pl.BlockSpec((D, E), lambda i: (0, 0)),
        ],
        out_specs=(blk, blk, blk, blk, blk, blk,
                   pl.BlockSpec((1, E), lambda i: (0, 0)),
                   pl.BlockSpec((1, 1), lambda i: (0, 0))),
        out_shape=out_shapes,
        scratch_shapes=[
            pltpu.VMEM((1, E), jnp.float32),
            pltpu.VMEM((1, E), jnp.float32),
            pltpu.VMEM((1, E), jnp.float32),
        ],
        interpret=interpret,
    )(x, wg)


def _slot_build_body(ti0_hbm, ti1_hbm, p0_hbm, p1_hbm, w0_hbm, w1_hbm,
                     cnt0_hbm, s2t_hbm, sw_hbm, ci0_hbm, ci1_hbm,
                     ti0_v, ti1_v, p0_v, p1_v, w0_v, w1_v, cnt0_v,
                     s2t_v, sw_v, ci0_v, ci1_v):
    wid = lax.axis_index("s") * 2 + lax.axis_index("c")

    @pl.when(wid == 0)
    def _():
        pltpu.sync_copy(ti0_hbm, ti0_v)
        pltpu.sync_copy(ti1_hbm, ti1_v)
        pltpu.sync_copy(p0_hbm, p0_v)
        pltpu.sync_copy(p1_hbm, p1_v)
        pltpu.sync_copy(w0_hbm, w0_v)
        pltpu.sync_copy(w1_hbm, w1_v)
        pltpu.sync_copy(cnt0_hbm, cnt0_v)

        zi = jnp.zeros((16,), jnp.int32)
        zf = jnp.zeros((16,), jnp.float32)

        def zloop(i, carry):
            s2t_v[pl.ds(i * 16, 16)] = zi
            sw_v[pl.ds(i * 16, 16)] = zf
            return carry

        lax.fori_loop(0, NSLOT_PAD // 16, zloop, 0)

        def tloop(i, carry):
            sl = pl.ds(i * 16, 16)
            t0 = ti0_v[sl]
            t1 = ti1_v[sl]
            pos0 = p0_v[sl]
            pos1 = p1_v[sl] + plsc.load_gather(cnt0_v, [t1])
            tok = i * 16 + lax.iota(jnp.int32, 16)
            k0 = pos0 < CAP
            k1 = pos1 < CAP
            d0 = t0 * CAP + pos0
            d1 = t1 * CAP + pos1
            d0c = jnp.where(k0, d0, 0)
            d1c = jnp.where(k1, d1, 0)
            plsc.store_scatter(s2t_v, [d0c], tok, mask=k0)
            plsc.store_scatter(sw_v, [d0c], w0_v[sl], mask=k0)
            plsc.store_scatter(s2t_v, [d1c], tok, mask=k1)
            plsc.store_scatter(sw_v, [d1c], w1_v[sl], mask=k1)
            ci0_v[sl] = jnp.where(k0, d0, NSLOT)
            ci1_v[sl] = jnp.where(k1, d1, NSLOT)
            return carry

        lax.fori_loop(0, T // 16, tloop, 0)

        pltpu.sync_copy(s2t_v, s2t_hbm)
        pltpu.sync_copy(sw_v, sw_hbm)
        pltpu.sync_copy(ci0_v, ci0_hbm)
        pltpu.sync_copy(ci1_v, ci1_hbm)


def _dispatch_body(x_hbm, s2t_hbm, disp_hbm, idx_v, rows_v, sem):
    wid = lax.axis_index("s") * 2 + lax.axis_index("c")
    for c in range(ROWS_W // DCHUNK):
        base = wid * ROWS_W + c * DCHUNK
        pltpu.sync_copy(s2t_hbm.at[pl.ds(base, DCHUNK)], idx_v)
        pltpu.async_copy(x_hbm.at[idx_v], rows_v, sem).wait()
        pltpu.sync_copy(rows_v, disp_hbm.at[pl.ds(base, DCHUNK)])


def _combine_body(eo_hbm, ci0_hbm, ci1_hbm, y_hbm, i0_v, i1_v, a_v, b_v, sem):
    wid = lax.axis_index("s") * 2 + lax.axis_index("c")
    for c in range(TOK_W // CCHUNK):
        base = wid * TOK_W + c * CCHUNK
        pltpu.sync_copy(ci0_hbm.at[pl.ds(base, CCHUNK)], i0_v)
        pltpu.sync_copy(ci1_hbm.at[pl.ds(base, CCHUNK)], i1_v)
        cp0 = pltpu.async_copy(eo_hbm.at[i0_v], a_v, sem)
        cp1 = pltpu.async_copy(eo_hbm.at[i1_v], b_v, sem)
        cp0.wait()
        cp1.wait()

        def radd(r, carry):
            for j in range(D // 16):
                sl = pl.ds(j * 16, 16)
                a_v[r, sl] = a_v[r, sl] + b_v[r, sl]
            return carry

        lax.fori_loop(0, CCHUNK, radd, 0)
        pltpu.sync_copy(a_v, y_hbm.at[pl.ds(base, CCHUNK)])


@functools.lru_cache(maxsize=None)
def _sc_kernels():
    mesh = plsc.VectorSubcoreMesh(core_axis_name="c", subcore_axis_name="s")

    slot_build = pl.kernel(
        _slot_build_body,
        out_type=(
            jax.ShapeDtypeStruct((NSLOT_PAD,), jnp.int32),
            jax.ShapeDtypeStruct((NSLOT_PAD,), jnp.float32),
            jax.ShapeDtypeStruct((T,), jnp.int32),
            jax.ShapeDtypeStruct((T,), jnp.int32),
        ),
        mesh=mesh,
        compiler_params=pltpu.CompilerParams(needs_layout_passes=False),
        scratch_types=[
            pltpu.VMEM((T,), jnp.int32),
            pltpu.VMEM((T,), jnp.int32),
            pltpu.VMEM((T,), jnp.int32),
            pltpu.VMEM((T,), jnp.int32),
            pltpu.VMEM((T,), jnp.float32),
            pltpu.VMEM((T,), jnp.float32),
            pltpu.VMEM((16,), jnp.int32),
            pltpu.VMEM((NSLOT_PAD,), jnp.int32),
            pltpu.VMEM((NSLOT_PAD,), jnp.float32),
            pltpu.VMEM((T,), jnp.int32),
            pltpu.VMEM((T,), jnp.int32),
        ],
    )

    dispatch = pl.kernel(
        _dispatch_body,
        out_type=jax.ShapeDtypeStruct((NSLOT_PAD, D), jnp.float32),
        mesh=mesh,
        compiler_params=pltpu.CompilerParams(needs_layout_passes=False),
        scratch_types=[
            pltpu.VMEM((DCHUNK,), jnp.int32),
            pltpu.VMEM((DCHUNK, D), jnp.float32),
            pltpu.SemaphoreType.DMA,
        ],
    )

    combine = pl.kernel(
        _combine_body,
        out_type=jax.ShapeDtypeStruct((T, D), jnp.float32),
        mesh=mesh,
        compiler_params=pltpu.CompilerParams(needs_layout_passes=False),
        scratch_types=[
            pltpu.VMEM((CCHUNK,), jnp.int32),
            pltpu.VMEM((CCHUNK,), jnp.int32),
            pltpu.VMEM((CCHUNK, D), jnp.float32),
            pltpu.VMEM((CCHUNK, D), jnp.float32),
            pltpu.SemaphoreType.DMA,
        ],
    )
    return slot_build, dispatch, combine


def _bmm_body(sw_ref, disp_ref, wg_ref, wu_ref, wd_ref, out_ref):
    xb = disp_ref[...].astype(jnp.bfloat16)
    g = jnp.dot(xb, wg_ref[0], preferred_element_type=jnp.float32)
    u = jnp.dot(xb, wu_ref[0], preferred_element_type=jnp.float32)
    h = g * jax.nn.sigmoid(g) * u
    o = jnp.dot(h.astype(jnp.bfloat16), wd_ref[0],
                preferred_element_type=jnp.float32)
    out_ref[...] = o * sw_ref[...]


def _bmm(sw, disp, w_gate, w_up, w_down, interpret=False):
    nblk = NSLOT_PAD // 128
    eidx = lambda i: (jnp.minimum(i // 8, E - 1), 0, 0)
    return pl.pallas_call(
        _bmm_body,
        grid=(nblk,),
        in_specs=[
            pl.BlockSpec((128, 1), lambda i: (i, 0)),
            pl.BlockSpec((128, D), lambda i: (i, 0)),
            pl.BlockSpec((1, D, I), eidx),
            pl.BlockSpec((1, D, I), eidx),
            pl.BlockSpec((1, I, D), eidx),
        ],
        out_specs=pl.BlockSpec((128, D), lambda i: (i, 0)),
        out_shape=jax.ShapeDtypeStruct((NSLOT_PAD, D), jnp.float32),
        interpret=interpret,
    )(sw, disp, w_gate, w_up, w_down)


@jax.jit
def kernel(x, wg, w_gate, w_up, w_down):
    slot_build, dispatch, combine = _sc_kernels()
    ti0, ti1, p0, p1, w0, w1, cnt0, laux = _router(x, wg)
    s2t, sw, ci0, ci1 = slot_build(
        ti0.reshape(T), ti1.reshape(T), p0.reshape(T), p1.reshape(T),
        w0.reshape(T), w1.reshape(T),
        jnp.concatenate([cnt0.reshape(E), jnp.zeros((16 - E,), jnp.int32)]))
    disp = dispatch(x, s2t)
    eo = _bmm(sw.reshape(NSLOT_PAD, 1), disp,
              w_gate.astype(jnp.bfloat16), w_up.astype(jnp.bfloat16),
              w_down.astype(jnp.bfloat16))
    y = combine(eo, ci0, ci1)
    return y, laux.reshape(())


# speedup vs baseline: 1.1040x; 1.1008x over previous
import functools

import jax
import jax.numpy as jnp
from jax import lax
from jax.experimental import pallas as pl
from jax.experimental.pallas import tpu as pltpu
from jax.experimental.pallas import tpu_sc as plsc

T = 4096
D = 1024
E = 8
I = 512
CAP = 1024
NSLOT = E * CAP
NSLOT_PAD = 8448
NB = 8
BT = T // NB
NW = 32
ROWS_W = NSLOT_PAD // NW
DCHUNK = 88
TOK_W = T // NW
CCHUNK = 16


def _router_body(x_ref, wg_ref, ti0_ref, ti1_ref, p0_ref, p1_ref,
                 w0_ref, w1_ref, cnt0_ref, laux_ref,
                 off0, off1, me_acc):
    b = pl.program_id(0)

    @pl.when(b == 0)
    def _init():
        off0[...] = jnp.zeros_like(off0)
        off1[...] = jnp.zeros_like(off1)
        me_acc[...] = jnp.zeros_like(me_acc)

    xb = x_ref[...]
    wg = wg_ref[...]
    logits = jnp.dot(xb, wg, preferred_element_type=jnp.float32)
    m = jnp.max(logits, axis=1, keepdims=True)
    ex = jnp.exp(logits - m)
    gates = ex / jnp.sum(ex, axis=1, keepdims=True)

    lane = lax.broadcasted_iota(jnp.int32, (BT, E), 1)
    v0 = jnp.max(gates, axis=1, keepdims=True)
    i0 = jnp.min(jnp.where(gates == v0, lane, E), axis=1, keepdims=True)
    g1 = jnp.where(lane == i0, -jnp.inf, gates)
    v1 = jnp.max(g1, axis=1, keepdims=True)
    i1 = jnp.min(jnp.where(g1 == v1, lane, E), axis=1, keepdims=True)
    denom = v0 + v1 + 1e-9
    mask0 = (lane == i0).astype(jnp.float32)
    mask1 = (lane == i1).astype(jnp.float32)

    tri = (lax.broadcasted_iota(jnp.int32, (BT, BT), 0)
           >= lax.broadcasted_iota(jnp.int32, (BT, BT), 1)).astype(jnp.float32)
    c0 = jnp.dot(tri, mask0, preferred_element_type=jnp.float32)
    c1 = jnp.dot(tri, mask1, preferred_element_type=jnp.float32)

    o0 = off0[...]
    o1 = off1[...]
    pos0 = jnp.sum(mask0 * (c0 - 1.0 + o0), axis=1, keepdims=True)
    pos1 = jnp.sum(mask1 * (c1 - 1.0 + o1), axis=1, keepdims=True)
    new_off0 = o0 + jnp.sum(mask0, axis=0, keepdims=True)
    off0[...] = new_off0
    off1[...] = o1 + jnp.sum(mask1, axis=0, keepdims=True)
    me_acc[...] = me_acc[...] + jnp.sum(gates, axis=0, keepdims=True)

    ti0_ref[...] = i0.astype(jnp.int32).reshape(1, BT, 1)
    ti1_ref[...] = i1.astype(jnp.int32).reshape(1, BT, 1)
    p0_ref[...] = pos0.astype(jnp.int32).reshape(1, BT, 1)
    p1_ref[...] = pos1.astype(jnp.int32).reshape(1, BT, 1)
    w0_ref[...] = (v0 / denom).reshape(1, BT, 1)
    w1_ref[...] = (v1 / denom).reshape(1, BT, 1)
    cnt0_ref[...] = new_off0.astype(jnp.int32)

    @pl.when(b == NB - 1)
    def _fin():
        me = me_acc[...] / float(T)
        ce = new_off0 / float(T)
        laux_ref[...] = jnp.sum(me * ce).reshape(1, 1) * float(E)


def _router(x, wg, interpret=False):
    out_shapes = (
        jax.ShapeDtypeStruct((NB, BT, 1), jnp.int32),
        jax.ShapeDtypeStruct((NB, BT, 1), jnp.int32),
        jax.ShapeDtypeStruct((NB, BT, 1), jnp.int32),
        jax.ShapeDtypeStruct((NB, BT, 1), jnp.int32),
        jax.ShapeDtypeStruct((NB, BT, 1), jnp.float32),
        jax.ShapeDtypeStruct((NB, BT, 1), jnp.float32),
        jax.ShapeDtypeStruct((1, E), jnp.int32),
        jax.ShapeDtypeStruct((1, 1), jnp.float32),
    )
    blk = pl.BlockSpec((1, BT, 1), lambda i: (i, 0, 0))
    return pl.pallas_call(
        _router_body,
        grid=(NB,),
        in_specs=[
            pl.BlockSpec((BT, D), lambda i: (i, 0)),
            pl.BlockSpec((D, E), lambda i: (0, 0)),
        ],
        out_specs=(blk, blk, blk, blk, blk, blk,
                   pl.BlockSpec((1, E), lambda i: (0, 0)),
                   pl.BlockSpec((1, 1), lambda i: (0, 0))),
        out_shape=out_shapes,
        scratch_shapes=[
            pltpu.VMEM((1, E), jnp.float32),
            pltpu.VMEM((1, E), jnp.float32),
            pltpu.VMEM((1, E), jnp.float32),
        ],
        interpret=interpret,
    )(x, wg)


def _slot_build_body(ti0_hbm, ti1_hbm, p0_hbm, p1_hbm, w0_hbm, w1_hbm,
                     cnt0_hbm, s2t_hbm, sw_hbm, ci0_hbm, ci1_hbm,
                     ti0_v, ti1_v, p0_v, p1_v, w0_v, w1_v, cnt0_v,
                     s2t_v, sw_v, ci0_v, ci1_v):
    wid = lax.axis_index("s") * 2 + lax.axis_index("c")

    @pl.when(wid == 0)
    def _():
        pltpu.sync_copy(ti0_hbm, ti0_v)
        pltpu.sync_copy(ti1_hbm, ti1_v)
        pltpu.sync_copy(p0_hbm, p0_v)
        pltpu.sync_copy(p1_hbm, p1_v)
        pltpu.sync_copy(w0_hbm, w0_v)
        pltpu.sync_copy(w1_hbm, w1_v)
        pltpu.sync_copy(cnt0_hbm, cnt0_v)

        zi = jnp.zeros((16,), jnp.int32)
        zf = jnp.zeros((16,), jnp.float32)

        def zloop(i, carry):
            s2t_v[pl.ds(i * 16, 16)] = zi
            sw_v[pl.ds(i * 16, 16)] = zf
            return carry

        lax.fori_loop(0, NSLOT_PAD // 16, zloop, 0)

        def tloop(i, carry):
            sl = pl.ds(i * 16, 16)
            t0 = ti0_v[sl]
            t1 = ti1_v[sl]
            pos0 = p0_v[sl]
            pos1 = p1_v[sl] + plsc.load_gather(cnt0_v, [t1])
            tok = i * 16 + lax.iota(jnp.int32, 16)
            k0 = pos0 < CAP
            k1 = pos1 < CAP
            d0 = t0 * CAP + pos0
            d1 = t1 * CAP + pos1
            d0c = jnp.where(k0, d0, 0)
            d1c = jnp.where(k1, d1, 0)
            plsc.store_scatter(s2t_v, [d0c], tok, mask=k0)
            plsc.store_scatter(sw_v, [d0c], w0_v[sl], mask=k0)
            plsc.store_scatter(s2t_v, [d1c], tok, mask=k1)
            plsc.store_scatter(sw_v, [d1c], w1_v[sl], mask=k1)
            ci0_v[sl] = jnp.where(k0, d0, NSLOT)
            ci1_v[sl] = jnp.where(k1, d1, NSLOT)
            return carry

        lax.fori_loop(0, T // 16, tloop, 0)

        pltpu.sync_copy(s2t_v, s2t_hbm)
        pltpu.sync_copy(sw_v, sw_hbm)
        pltpu.sync_copy(ci0_v, ci0_hbm)
        pltpu.sync_copy(ci1_v, ci1_hbm)


DCHUNKS = (56, 56, 56, 56, 40)


def _dispatch_body(x_hbm, s2t_hbm, disp_hbm, idx_v, rows0_v, rows1_v,
                   sem0, sem1):
    wid = lax.axis_index("s") * 2 + lax.axis_index("c")
    base = wid * ROWS_W
    pltpu.sync_copy(s2t_hbm.at[pl.ds(base, ROWS_W)], idx_v)
    bufs = (rows0_v, rows1_v)
    sems = (sem0, sem1)
    offs = [0]
    for n in DCHUNKS[:-1]:
        offs.append(offs[-1] + n)

    def start(c):
        n = DCHUNKS[c]
        src = x_hbm.at[idx_v.at[pl.ds(offs[c], n)]]
        return pltpu.async_copy(src, bufs[c % 2].at[pl.ds(0, n)], sems[c % 2])

    cps = {0: start(0)}
    for c in range(len(DCHUNKS)):
        cps.pop(c).wait()
        if c + 1 < len(DCHUNKS):
            cps[c + 1] = start(c + 1)
        n = DCHUNKS[c]
        pltpu.sync_copy(bufs[c % 2].at[pl.ds(0, n)],
                        disp_hbm.at[pl.ds(base + offs[c], n)])


def _combine_body(eo_hbm, ci0_hbm, ci1_hbm, y_hbm, i0_v, i1_v,
                  a0_v, b0_v, a1_v, b1_v, sem0, sem1):
    wid = lax.axis_index("s") * 2 + lax.axis_index("c")
    tbase = wid * TOK_W
    pltpu.sync_copy(ci0_hbm.at[pl.ds(tbase, TOK_W)], i0_v)
    pltpu.sync_copy(ci1_hbm.at[pl.ds(tbase, TOK_W)], i1_v)
    abufs = (a0_v, a1_v)
    bbufs = (b0_v, b1_v)
    sems = (sem0, sem1)
    nchunk = TOK_W // CCHUNK

    def start(c):
        s = c % 2
        idx0 = i0_v[pl.ds(c * CCHUNK, CCHUNK)]
        idx1 = i1_v[pl.ds(c * CCHUNK, CCHUNK)]
        cpa = pltpu.async_copy(eo_hbm.at[idx0], abufs[s], sems[s])
        cpb = pltpu.async_copy(eo_hbm.at[idx1], bbufs[s], sems[s])
        return cpa, cpb

    cps = {0: start(0)}
    for c in range(nchunk):
        s = c % 2
        cpa, cpb = cps.pop(c)
        cpa.wait()
        cpb.wait()
        if c + 1 < nchunk:
            cps[c + 1] = start(c + 1)
        a_v = abufs[s]
        b_v = bbufs[s]

        def radd(r, carry):
            for j in range(D // 16):
                sl = pl.ds(j * 16, 16)
                a_v[r, sl] = a_v[r, sl] + b_v[r, sl]
            return carry

        lax.fori_loop(0, CCHUNK, radd, 0)
        pltpu.sync_copy(a_v, y_hbm.at[pl.ds(tbase + c * CCHUNK, CCHUNK)])


@functools.lru_cache(maxsize=None)
def _sc_kernels():
    mesh = plsc.VectorSubcoreMesh(core_axis_name="c", subcore_axis_name="s")

    slot_build = pl.kernel(
        _slot_build_body,
        out_type=(
            jax.ShapeDtypeStruct((NSLOT_PAD,), jnp.int32),
            jax.ShapeDtypeStruct((NSLOT_PAD,), jnp.float32),
            jax.ShapeDtypeStruct((T,), jnp.int32),
            jax.ShapeDtypeStruct((T,), jnp.int32),
        ),
        mesh=mesh,
        compiler_params=pltpu.CompilerParams(needs_layout_passes=False),
        scratch_types=[
            pltpu.VMEM((T,), jnp.int32),
            pltpu.VMEM((T,), jnp.int32),
            pltpu.VMEM((T,), jnp.int32),
            pltpu.VMEM((T,), jnp.int32),
            pltpu.VMEM((T,), jnp.float32),
            pltpu.VMEM((T,), jnp.float32),
            pltpu.VMEM((16,), jnp.int32),
            pltpu.VMEM((NSLOT_PAD,), jnp.int32),
            pltpu.VMEM((NSLOT_PAD,), jnp.float32),
            pltpu.VMEM((T,), jnp.int32),
            pltpu.VMEM((T,), jnp.int32),
        ],
    )

    dispatch = pl.kernel(
        _dispatch_body,
        out_type=jax.ShapeDtypeStruct((NSLOT_PAD, D), jnp.float32),
        mesh=mesh,
        compiler_params=pltpu.CompilerParams(needs_layout_passes=False),
        scratch_types=[
            pltpu.VMEM((ROWS_W,), jnp.int32),
            pltpu.VMEM((56, D), jnp.float32),
            pltpu.VMEM((56, D), jnp.float32),
            pltpu.SemaphoreType.DMA,
            pltpu.SemaphoreType.DMA,
        ],
    )

    combine = pl.kernel(
        _combine_body,
        out_type=jax.ShapeDtypeStruct((T, D), jnp.float32),
        mesh=mesh,
        compiler_params=pltpu.CompilerParams(needs_layout_passes=False),
        scratch_types=[
            pltpu.VMEM((TOK_W,), jnp.int32),
            pltpu.VMEM((TOK_W,), jnp.int32),
            pltpu.VMEM((CCHUNK, D), jnp.float32),
            pltpu.VMEM((CCHUNK, D), jnp.float32),
            pltpu.VMEM((CCHUNK, D), jnp.float32),
            pltpu.VMEM((CCHUNK, D), jnp.float32),
            pltpu.SemaphoreType.DMA,
            pltpu.SemaphoreType.DMA,
        ],
    )
    return slot_build, dispatch, combine


def _bmm_body(sw_ref, disp_ref, wg_ref, wu_ref, wd_ref, out_ref):
    xb = disp_ref[...]
    g = jnp.dot(xb, wg_ref[0], preferred_element_type=jnp.float32)
    u = jnp.dot(xb, wu_ref[0], preferred_element_type=jnp.float32)
    h = g * jax.nn.sigmoid(g) * u
    o = jnp.dot(h, wd_ref[0], preferred_element_type=jnp.float32)
    out_ref[...] = o * sw_ref[...]


def _bmm(sw, disp, w_gate, w_up, w_down, interpret=False):
    nblk = NSLOT_PAD // 128
    eidx = lambda i: (jnp.minimum(i // 8, E - 1), 0, 0)
    return pl.pallas_call(
        _bmm_body,
        grid=(nblk,),
        in_specs=[
            pl.BlockSpec((128, 1), lambda i: (i, 0)),
            pl.BlockSpec((128, D), lambda i: (i, 0)),
            pl.BlockSpec((1, D, I), eidx),
            pl.BlockSpec((1, D, I), eidx),
            pl.BlockSpec((1, I, D), eidx),
        ],
        out_specs=pl.BlockSpec((128, D), lambda i: (i, 0)),
        out_shape=jax.ShapeDtypeStruct((NSLOT_PAD, D), jnp.float32),
        interpret=interpret,
    )(sw, disp, w_gate, w_up, w_down)


@jax.jit
def kernel(x, wg, w_gate, w_up, w_down):
    slot_build, dispatch, combine = _sc_kernels()
    ti0, ti1, p0, p1, w0, w1, cnt0, laux = _router(x, wg)
    s2t, sw, ci0, ci1 = slot_build(
        ti0.reshape(T), ti1.reshape(T), p0.reshape(T), p1.reshape(T),
        w0.reshape(T), w1.reshape(T),
        jnp.concatenate([cnt0.reshape(E), jnp.zeros((16 - E,), jnp.int32)]))
    disp = dispatch(x, s2t)
    eo = _bmm(sw.reshape(NSLOT_PAD, 1), disp, w_gate, w_up, w_down)
    y = combine(eo, ci0, ci1)
    return y, laux.reshape(())


# speedup vs baseline: 1.4370x; 1.3016x over previous
import functools

import jax
import jax.numpy as jnp
from jax import lax
from jax.experimental import pallas as pl
from jax.experimental.pallas import tpu as pltpu
from jax.experimental.pallas import tpu_sc as plsc

T = 4096
D = 1024
E = 8
I = 512
CAP = 1024
NSLOT = E * CAP
ZROW = NSLOT
NROW = 8320
DUMP = NROW
DISP_ROWS = 8448
NB = 8
BT = T // NB
NW = 32
TOK_W = T // NW
DCH = 32
CCH = 16


def _router_body(x_ref, wg_ref, d0_ref, d1_ref, ci0_ref, ci1_ref,
                 wk0_ref, wk1_ref, laux_ref,
                 ti0_s, ti1_s, p0_s, p1_s, w0_s, w1_s, off0, off1, me_acc):
    p = pl.program_id(0)
    b = pl.program_id(1)

    @pl.when(jnp.logical_and(p == 0, b == 0))
    def _init():
        off0[...] = jnp.zeros_like(off0)
        off1[...] = jnp.zeros_like(off1)
        me_acc[...] = jnp.zeros_like(me_acc)

    tok_sl = pl.ds(b * BT, BT)

    @pl.when(p == 0)
    def _pass0():
        xb = x_ref[...]
        wg = wg_ref[...]
        logits = jnp.dot(xb, wg, preferred_element_type=jnp.float32)
        m = jnp.max(logits, axis=1, keepdims=True)
        ex = jnp.exp(logits - m)
        gates = ex / jnp.sum(ex, axis=1, keepdims=True)

        lane = lax.broadcasted_iota(jnp.int32, (BT, E), 1)
        v0 = jnp.max(gates, axis=1, keepdims=True)
        i0 = jnp.min(jnp.where(gates == v0, lane, E), axis=1, keepdims=True)
        g1 = jnp.where(lane == i0, -jnp.inf, gates)
        v1 = jnp.max(g1, axis=1, keepdims=True)
        i1 = jnp.min(jnp.where(g1 == v1, lane, E), axis=1, keepdims=True)
        denom = v0 + v1 + 1e-9
        mask0 = (lane == i0).astype(jnp.float32)
        mask1 = (lane == i1).astype(jnp.float32)

        tri = (lax.broadcasted_iota(jnp.int32, (BT, BT), 0)
               >= lax.broadcasted_iota(jnp.int32, (BT, BT), 1)
               ).astype(jnp.float32)
        c0 = jnp.dot(tri, mask0, preferred_element_type=jnp.float32)
        c1 = jnp.dot(tri, mask1, preferred_element_type=jnp.float32)

        o0 = off0[...]
        o1 = off1[...]
        pos0 = jnp.sum(mask0 * (c0 - 1.0 + o0), axis=1, keepdims=True)
        pos1 = jnp.sum(mask1 * (c1 - 1.0 + o1), axis=1, keepdims=True)
        off0[...] = o0 + jnp.sum(mask0, axis=0, keepdims=True)
        off1[...] = o1 + jnp.sum(mask1, axis=0, keepdims=True)
        me_acc[...] = me_acc[...] + jnp.sum(gates, axis=0, keepdims=True)

        ti0_s[tok_sl, :] = i0
        ti1_s[tok_sl, :] = i1
        p0_s[tok_sl, :] = pos0.astype(jnp.int32)
        p1_s[tok_sl, :] = pos1.astype(jnp.int32)
        w0_s[tok_sl, :] = v0 / denom
        w1_s[tok_sl, :] = v1 / denom

    @pl.when(p == 1)
    def _pass1():
        i0 = ti0_s[tok_sl, :]
        i1 = ti1_s[tok_sl, :]
        pos0 = p0_s[tok_sl, :]
        w0 = w0_s[tok_sl, :]
        w1 = w1_s[tok_sl, :]
        lane = lax.broadcasted_iota(jnp.int32, (BT, E), 1)
        mask1 = (lane == i1).astype(jnp.float32)
        cnt_g = jnp.sum(mask1 * off0[...], axis=1, keepdims=True)
        pos1 = p1_s[tok_sl, :] + cnt_g.astype(jnp.int32)

        k0 = pos0 < CAP
        k1 = pos1 < CAP
        s0 = i0 * CAP + pos0
        s1 = i1 * CAP + pos1
        d0 = jnp.where(k0, s0, DUMP)
        d1 = jnp.where(k1, s1, DUMP)
        d0_ref[...] = d0.reshape(4, 4, DCH)
        d1_ref[...] = d1.reshape(4, 4, DCH)
        ci0_ref[...] = jnp.where(k0, s0, ZROW).reshape(1, BT, 1)
        ci1_ref[...] = jnp.where(k1, s1, ZROW).reshape(1, BT, 1)
        wk0_ref[...] = jnp.where(k0, w0, 0.0).reshape(1, BT, 1)
        wk1_ref[...] = jnp.where(k1, w1, 0.0).reshape(1, BT, 1)

        @pl.when(b == NB - 1)
        def _fin():
            me = me_acc[...] / float(T)
            ce = off0[...] / float(T)
            laux_ref[...] = jnp.sum(me * ce).reshape(1, 1) * float(E)


def _router(x, wg):
    out_shapes = (
        jax.ShapeDtypeStruct((NW, 4, DCH), jnp.int32),
        jax.ShapeDtypeStruct((NW, 4, DCH), jnp.int32),
        jax.ShapeDtypeStruct((NB, BT, 1), jnp.int32),
        jax.ShapeDtypeStruct((NB, BT, 1), jnp.int32),
        jax.ShapeDtypeStruct((NB, BT, 1), jnp.float32),
        jax.ShapeDtypeStruct((NB, BT, 1), jnp.float32),
        jax.ShapeDtypeStruct((1, 1), jnp.float32),
    )
    blk = pl.BlockSpec((1, BT, 1), lambda p, b: (b, 0, 0))
    dblk = pl.BlockSpec((4, 4, DCH), lambda p, b: (b, 0, 0))
    return pl.pallas_call(
        _router_body,
        grid=(2, NB),
        in_specs=[
            pl.BlockSpec((BT, D), lambda p, b: (b * (1 - p), 0)),
            pl.BlockSpec((D, E), lambda p, b: (0, 0)),
        ],
        out_specs=(dblk, dblk, blk, blk, blk, blk,
                   pl.BlockSpec((1, 1), lambda p, b: (0, 0))),
        out_shape=out_shapes,
        scratch_shapes=[
            pltpu.VMEM((T, 1), jnp.int32),
            pltpu.VMEM((T, 1), jnp.int32),
            pltpu.VMEM((T, 1), jnp.int32),
            pltpu.VMEM((T, 1), jnp.int32),
            pltpu.VMEM((T, 1), jnp.float32),
            pltpu.VMEM((T, 1), jnp.float32),
            pltpu.VMEM((1, E), jnp.float32),
            pltpu.VMEM((1, E), jnp.float32),
            pltpu.VMEM((1, E), jnp.float32),
        ],
    )(x, wg)


def _dispatch_body(x_hbm, d0_hbm, d1_hbm, disp_hbm,
                   d0_v, d1_v, xb0_v, xb1_v, z_v, sem0, sem1):
    wid = lax.axis_index("s") * 2 + lax.axis_index("c")
    tbase = wid * TOK_W

    pltpu.sync_copy(d0_hbm.at[wid], d0_v)
    pltpu.sync_copy(d1_hbm.at[wid], d1_v)

    zrow = jnp.zeros((16,), jnp.float32)

    def zloop(r, carry):
        for j in range(D // 16):
            z_v[r, pl.ds(j * 16, 16)] = zrow
        return carry

    lax.fori_loop(0, 4, zloop, 0)
    pltpu.sync_copy(z_v, disp_hbm.at[pl.ds(NSLOT + wid * 4, 4)])

    bufs = (xb0_v, xb1_v)
    sems = (sem0, sem1)
    nch = TOK_W // DCH

    def start(c):
        return pltpu.async_copy(
            x_hbm.at[pl.ds(tbase + c * DCH, DCH)], bufs[c % 2], sems[c % 2])

    cps = {0: start(0)}
    for c in range(nch):
        cps.pop(c).wait()
        if c + 1 < nch:
            cps[c + 1] = start(c + 1)
        pltpu.sync_copy(bufs[c % 2], disp_hbm.at[d0_v.at[c]])
        pltpu.sync_copy(bufs[c % 2], disp_hbm.at[d1_v.at[c]])


def _combine_body(eo_hbm, ci0_hbm, ci1_hbm, w0_hbm, w1_hbm, y_hbm,
                  i0_v, i1_v, w0_v, w1_v, a0_v, b0_v, a1_v, b1_v, sem0, sem1):
    wid = lax.axis_index("s") * 2 + lax.axis_index("c")
    tbase = wid * TOK_W
    pltpu.sync_copy(ci0_hbm.at[pl.ds(tbase, TOK_W)], i0_v)
    pltpu.sync_copy(ci1_hbm.at[pl.ds(tbase, TOK_W)], i1_v)
    pltpu.sync_copy(w0_hbm.at[pl.ds(tbase, TOK_W)], w0_v)
    pltpu.sync_copy(w1_hbm.at[pl.ds(tbase, TOK_W)], w1_v)
    abufs = (a0_v, a1_v)
    bbufs = (b0_v, b1_v)
    sems = (sem0, sem1)
    nch = TOK_W // CCH

    def start(c):
        s = c % 2
        idx0 = i0_v[pl.ds(c * CCH, CCH)]
        idx1 = i1_v[pl.ds(c * CCH, CCH)]
        cpa = pltpu.async_copy(eo_hbm.at[idx0], abufs[s], sems[s])
        cpb = pltpu.async_copy(eo_hbm.at[idx1], bbufs[s], sems[s])
        return cpa, cpb

    cps = {0: start(0)}
    for c in range(nch):
        s = c % 2
        cpa, cpb = cps.pop(c)
        cpa.wait()
        cpb.wait()
        if c + 1 < nch:
            cps[c + 1] = start(c + 1)
        a_v = abufs[s]
        b_v = bbufs[s]
        wv0 = w0_v[pl.ds(c * CCH, CCH)]
        wv1 = w1_v[pl.ds(c * CCH, CCH)]

        def jadd(j, carry):
            sl = pl.ds(j * 16, 16)
            for r in range(CCH):
                a_v[r, sl] = a_v[r, sl] * wv0[r] + b_v[r, sl] * wv1[r]
            return carry

        lax.fori_loop(0, D // 16, jadd, 0)
        pltpu.sync_copy(a_v, y_hbm.at[pl.ds(tbase + c * CCH, CCH)])


@functools.lru_cache(maxsize=None)
def _sc_kernels():
    mesh = plsc.VectorSubcoreMesh(core_axis_name="c", subcore_axis_name="s")

    dispatch = pl.kernel(
        _dispatch_body,
        out_type=jax.ShapeDtypeStruct((DISP_ROWS, D), jnp.float32),
        mesh=mesh,
        compiler_params=pltpu.CompilerParams(needs_layout_passes=False),
        scratch_types=[
            pltpu.VMEM((4, DCH), jnp.int32),
            pltpu.VMEM((4, DCH), jnp.int32),
            pltpu.VMEM((DCH, D), jnp.float32),
            pltpu.VMEM((DCH, D), jnp.float32),
            pltpu.VMEM((4, D), jnp.float32),
            pltpu.SemaphoreType.DMA,
            pltpu.SemaphoreType.DMA,
        ],
    )

    combine = pl.kernel(
        _combine_body,
        out_type=jax.ShapeDtypeStruct((T, D), jnp.float32),
        mesh=mesh,
        compiler_params=pltpu.CompilerParams(needs_layout_passes=False),
        scratch_types=[
            pltpu.VMEM((TOK_W,), jnp.int32),
            pltpu.VMEM((TOK_W,), jnp.int32),
            pltpu.VMEM((TOK_W,), jnp.float32),
            pltpu.VMEM((TOK_W,), jnp.float32),
            pltpu.VMEM((CCH, D), jnp.float32),
            pltpu.VMEM((CCH, D), jnp.float32),
            pltpu.VMEM((CCH, D), jnp.float32),
            pltpu.VMEM((CCH, D), jnp.float32),
            pltpu.SemaphoreType.DMA,
            pltpu.SemaphoreType.DMA,
        ],
    )
    return dispatch, combine


def _bmm_body(disp_ref, wg_ref, wu_ref, wd_ref, out_ref):
    xb = disp_ref[...]
    g = jnp.dot(xb, wg_ref[0], preferred_element_type=jnp.float32)
    u = jnp.dot(xb, wu_ref[0], preferred_element_type=jnp.float32)
    h = g * jax.nn.sigmoid(g) * u
    out_ref[...] = jnp.dot(h, wd_ref[0], preferred_element_type=jnp.float32)


def _bmm(disp, w_gate, w_up, w_down, interpret=False):
    nblk = NROW // 128
    eidx = lambda i: (jnp.minimum(i // 8, E - 1), 0, 0)
    return pl.pallas_call(
        _bmm_body,
        grid=(nblk,),
        in_specs=[
            pl.BlockSpec((128, D), lambda i: (i, 0)),
            pl.BlockSpec((1, D, I), eidx),
            pl.BlockSpec((1, D, I), eidx),
            pl.BlockSpec((1, I, D), eidx),
        ],
        out_specs=pl.BlockSpec((128, D), lambda i: (i, 0)),
        out_shape=jax.ShapeDtypeStruct((NROW, D), jnp.float32),
        interpret=interpret,
    )(disp, w_gate, w_up, w_down)


@jax.jit
def kernel(x, wg, w_gate, w_up, w_down):
    dispatch, combine = _sc_kernels()
    d0, d1, ci0, ci1, wk0, wk1, laux = _router(x, wg)
    disp = dispatch(x, d0, d1)
    eo = _bmm(disp, w_gate, w_up, w_down)
    y = combine(eo, ci0.reshape(T), ci1.reshape(T),
                wk0.reshape(T), wk1.reshape(T))
    return y, laux.reshape(())
